# baseline (speedup 1.0000x reference)
import jax
import jax.numpy as jnp
from jax.experimental import pallas as pl
from jax.experimental.pallas import tpu as pltpu

_N = 32
_R = 3 * 512
_W = 512


def _write_x_body(x_ref, o_ref):
    o_ref[...] = x_ref[...]


def _shift_body(t_ref, o_alias_ref, o_ref):
    o_ref[...] = t_ref[...]


def kernel(x, tensors):
    x2 = x.reshape(_R, _W)
    t2 = tensors.reshape(_N * _R, _W)
    staged = pl.pallas_call(
        _write_x_body,
        grid=(1,),
        in_specs=[pl.BlockSpec((_R, _W), lambda i: (0, 0))],
        out_specs=pl.BlockSpec((_R, _W), lambda i: (_N - 1, 0)),
        out_shape=jax.ShapeDtypeStruct((_N * _R, _W), jnp.float32),
    )(x2)
    out = pl.pallas_call(
        _shift_body,
        grid=(_N - 1,),
        in_specs=[
            pl.BlockSpec((_R, _W), lambda i: (i + 1, 0)),
            pl.BlockSpec(memory_space=pl.ANY),
        ],
        out_specs=pl.BlockSpec((_R, _W), lambda i: (i, 0)),
        out_shape=jax.ShapeDtypeStruct((_N * _R, _W), jnp.float32),
        input_output_aliases={1: 0},
    )(t2, staged)
    return out.reshape(tensors.shape)
